# baseline (speedup 1.0000x reference)
import functools

import jax
import jax.numpy as jnp
from jax import lax
from jax.experimental import pallas as pl
from jax.experimental.pallas import tpu as pltpu
from jax.experimental.pallas import tpu_sc as plsc

BATCH = 16384
EMBED = 16
NC = 2
NS = 16
L = 16
NW = NC * NS
BPW = BATCH // NW
CQ = 8
NCHUNK = BPW // CQ
W = 128
CW = CQ * W
DEPTH = 3


def _mf_body(user_hbm, item_hbm, ut_hbm, it_hbm, out_hbm,
             uidx_v, iidx_v, ucols_v, icols_v, out_v, sems):
    wid = lax.axis_index("s") * NC + lax.axis_index("c")
    base = wid * BPW
    pltpu.sync_copy(user_hbm.at[pl.ds(base, BPW)], uidx_v.at[pl.ds(0, BPW)])
    pltpu.sync_copy(item_hbm.at[pl.ds(base, BPW)], iidx_v.at[pl.ds(0, BPW)])

    iot = lax.iota(jnp.int32, L)

    def fire(k, p):
        uvec = uidx_v[pl.ds(k * CQ, L)]
        ivec = iidx_v[pl.ds(k * CQ, L)]
        ustart = uvec & ~127
        istart = ivec & ~127
        sem = sems.at[p]
        for j in range(CQ):
            dst = pl.ds(p * CW + j * W, W)
            pltpu.async_copy(
                ut_hbm.at[:, pl.ds(pl.multiple_of(ustart[j], 128), W)],
                ucols_v.at[:, dst], sem)
            pltpu.async_copy(
                it_hbm.at[:, pl.ds(pl.multiple_of(istart[j], 128), W)],
                icols_v.at[:, dst], sem)

    def drain(p):
        sem = sems.at[p]
        pltpu.make_async_copy(
            ut_hbm.at[:, pl.ds(0, CW)], ucols_v.at[:, pl.ds(0, CW)], sem).wait()
        pltpu.make_async_copy(
            it_hbm.at[:, pl.ds(0, CW)], icols_v.at[:, pl.ds(0, CW)], sem).wait()

    def comp(k, p):
        uvec = uidx_v[pl.ds(k * CQ, L)]
        ivec = iidx_v[pl.ds(k * CQ, L)]
        cbase = p * CW + (iot & (CQ - 1)) * W
        ucol = cbase + (uvec & 127)
        icol = cbase + (ivec & 127)
        acc = jnp.zeros((L,), jnp.float32)
        for d in range(EMBED):
            drow = jnp.full((L,), d, jnp.int32)
            uu = plsc.load_gather(ucols_v, [drow, ucol])
            ii = plsc.load_gather(icols_v, [drow, icol])
            acc = acc + uu * ii
        out_v[pl.ds(k * CQ, L)] = acc

    for p in range(DEPTH):
        fire(p, p)

    def step(g, carry):
        p = lax.rem(g, DEPTH)
        drain(p)
        comp(g, p)
        fire(g + DEPTH, p)
        return carry

    lax.fori_loop(0, NCHUNK - DEPTH, step, 0)
    for k in range(NCHUNK - DEPTH, NCHUNK):
        p = k % DEPTH
        drain(p)
        comp(k, p)

    pltpu.sync_copy(out_v.at[pl.ds(0, BPW)], out_hbm.at[pl.ds(base, BPW)])


def kernel(user, item, U, I):
    user = user.astype(jnp.int32)
    item = item.astype(jnp.int32)
    Ut = U.T
    It = I.T
    mesh = plsc.VectorSubcoreMesh(core_axis_name="c", subcore_axis_name="s")
    k = functools.partial(
        pl.kernel,
        out_type=jax.ShapeDtypeStruct((BATCH,), jnp.float32),
        mesh=mesh,
        compiler_params=pltpu.CompilerParams(
            needs_layout_passes=False, use_tc_tiling_on_sc=False
        ),
        scratch_types=[
            pltpu.VMEM((BPW + L,), jnp.int32),
            pltpu.VMEM((BPW + L,), jnp.int32),
            pltpu.VMEM((EMBED, DEPTH * CW), jnp.float32),
            pltpu.VMEM((EMBED, DEPTH * CW), jnp.float32),
            pltpu.VMEM((BPW + L,), jnp.float32),
            pltpu.SemaphoreType.DMA((DEPTH,)),
        ],
    )(_mf_body)
    return k(user, item, Ut, It)


# speedup vs baseline: 21.8458x; 21.8458x over previous
import functools

import jax
import jax.numpy as jnp
from jax import lax
from jax.experimental import pallas as pl
from jax.experimental.pallas import tpu as pltpu
from jax.experimental.pallas import tpu_sc as plsc

BATCH = 16384
EMBED = 16
NC = 2
NS = 16
L = 16
NW = NC * NS
BPW = BATCH // NW
CQ = 8
NCHUNK = BPW // CQ
W = 128
CW = CQ * W
DEPTH = 3


def _mf_body(user_hbm, item_hbm, ut_hbm, it_hbm, out_hbm,
             uidx_v, iidx_v, ucols_v, icols_v, out_v, sems):
    wid = lax.axis_index("s") * NC + lax.axis_index("c")
    base = wid * BPW
    pltpu.sync_copy(user_hbm.at[pl.ds(base, BPW)], uidx_v.at[pl.ds(0, BPW)])
    pltpu.sync_copy(item_hbm.at[pl.ds(base, BPW)], iidx_v.at[pl.ds(0, BPW)])

    iot = lax.iota(jnp.int32, L)

    def fire(k, p):
        uvec = uidx_v[pl.ds(k * CQ, L)]
        ivec = iidx_v[pl.ds(k * CQ, L)]
        ustart = uvec & ~127
        istart = ivec & ~127
        sem = sems.at[p]
        for j in range(CQ):
            dst = pl.ds(p * CW + j * W, W)
            pltpu.async_copy(
                ut_hbm.at[:, pl.ds(pl.multiple_of(ustart[j], 128), W)],
                ucols_v.at[:, dst], sem)
            pltpu.async_copy(
                it_hbm.at[:, pl.ds(pl.multiple_of(istart[j], 128), W)],
                icols_v.at[:, dst], sem)

    def drain(p):
        sem = sems.at[p]
        pltpu.make_async_copy(
            ut_hbm.at[:, pl.ds(0, CW)], ucols_v.at[:, pl.ds(0, CW)], sem).wait()
        pltpu.make_async_copy(
            it_hbm.at[:, pl.ds(0, CW)], icols_v.at[:, pl.ds(0, CW)], sem).wait()

    def comp(k, p):
        uvec = uidx_v[pl.ds(k * CQ, L)]
        ivec = iidx_v[pl.ds(k * CQ, L)]
        cbase = p * CW + (iot & (CQ - 1)) * W
        ucol = cbase + (uvec & 127)
        icol = cbase + (ivec & 127)
        acc = jnp.zeros((L,), jnp.float32)
        for d in range(EMBED):
            drow = jnp.full((L,), d, jnp.int32)
            uu = plsc.load_gather(ucols_v, [drow, ucol])
            ii = plsc.load_gather(icols_v, [drow, icol])
            acc = acc + uu * ii
        out_v[pl.ds(k * CQ, L)] = acc

    for p in range(DEPTH):
        fire(p, p)

    def step(g, carry):
        p = lax.rem(g, DEPTH)
        drain(p)
        comp(g, p)
        fire(g + DEPTH, p)
        return carry

    lax.fori_loop(0, NCHUNK - DEPTH, step, 0)
    for k in range(NCHUNK - DEPTH, NCHUNK):
        p = k % DEPTH
        drain(p)
        comp(k, p)

    pltpu.sync_copy(out_v.at[pl.ds(0, BPW)], out_hbm.at[pl.ds(base, BPW)])


def kernel(user, item, U, I):
    user = user.astype(jnp.int32)
    item = item.astype(jnp.int32)
    Ut = U.T
    It = I.T
    mesh = plsc.VectorSubcoreMesh(core_axis_name="c", subcore_axis_name="s")
    k = functools.partial(
        pl.kernel,
        out_type=jax.ShapeDtypeStruct((BATCH,), jnp.float32),
        mesh=mesh,
        compiler_params=pltpu.CompilerParams(
            needs_layout_passes=False, use_tc_tiling_on_sc=True
        ),
        scratch_types=[
            pltpu.VMEM((BPW + L,), jnp.int32),
            pltpu.VMEM((BPW + L,), jnp.int32),
            pltpu.VMEM((EMBED, DEPTH * CW), jnp.float32),
            pltpu.VMEM((EMBED, DEPTH * CW), jnp.float32),
            pltpu.VMEM((BPW + L,), jnp.float32),
            pltpu.SemaphoreType.DMA((DEPTH,)),
        ],
    )(_mf_body)
    return k(user, item, Ut, It)


# speedup vs baseline: 23.0411x; 1.0547x over previous
import functools

import jax
import jax.numpy as jnp
from jax import lax
from jax.experimental import pallas as pl
from jax.experimental.pallas import tpu as pltpu
from jax.experimental.pallas import tpu_sc as plsc

BATCH = 16384
EMBED = 16
NC = 2
NS = 16
L = 16
NW = NC * NS
BPW = BATCH // NW
CQ = 4
NCHUNK = BPW // CQ
W = 128
CW = CQ * W
DEPTH = 6


def _mf_body(user_hbm, item_hbm, ut_hbm, it_hbm, out_hbm,
             uidx_v, iidx_v, ucols_v, icols_v, out_v, sems):
    wid = lax.axis_index("s") * NC + lax.axis_index("c")
    base = wid * BPW
    pltpu.sync_copy(user_hbm.at[pl.ds(base, BPW)], uidx_v.at[pl.ds(0, BPW)])
    pltpu.sync_copy(item_hbm.at[pl.ds(base, BPW)], iidx_v.at[pl.ds(0, BPW)])

    iot = lax.iota(jnp.int32, L)

    def fire(k, p):
        uvec = uidx_v[pl.ds(k * CQ, L)]
        ivec = iidx_v[pl.ds(k * CQ, L)]
        ustart = uvec & ~127
        istart = ivec & ~127
        sem = sems.at[p]
        for j in range(CQ):
            dst = pl.ds(p * CW + j * W, W)
            pltpu.async_copy(
                ut_hbm.at[:, pl.ds(pl.multiple_of(ustart[j], 128), W)],
                ucols_v.at[:, dst], sem)
            pltpu.async_copy(
                it_hbm.at[:, pl.ds(pl.multiple_of(istart[j], 128), W)],
                icols_v.at[:, dst], sem)

    def drain(p):
        sem = sems.at[p]
        pltpu.make_async_copy(
            ut_hbm.at[:, pl.ds(0, CW)], ucols_v.at[:, pl.ds(0, CW)], sem).wait()
        pltpu.make_async_copy(
            it_hbm.at[:, pl.ds(0, CW)], icols_v.at[:, pl.ds(0, CW)], sem).wait()

    def comp(k, p):
        uvec = uidx_v[pl.ds(k * CQ, L)]
        ivec = iidx_v[pl.ds(k * CQ, L)]
        cbase = p * CW + (iot & (CQ - 1)) * W
        ucol = cbase + (uvec & 127)
        icol = cbase + (ivec & 127)
        acc = jnp.zeros((L,), jnp.float32)
        for d in range(EMBED):
            drow = jnp.full((L,), d, jnp.int32)
            uu = plsc.load_gather(ucols_v, [drow, ucol])
            ii = plsc.load_gather(icols_v, [drow, icol])
            acc = acc + uu * ii
        out_v[pl.ds(k * CQ, L)] = acc

    for p in range(DEPTH):
        fire(p, p)

    def step(g, carry):
        p = lax.rem(g, DEPTH)
        drain(p)
        comp(g, p)
        fire(g + DEPTH, p)
        return carry

    lax.fori_loop(0, NCHUNK - DEPTH, step, 0)
    for k in range(NCHUNK - DEPTH, NCHUNK):
        p = k % DEPTH
        drain(p)
        comp(k, p)

    pltpu.sync_copy(out_v.at[pl.ds(0, BPW)], out_hbm.at[pl.ds(base, BPW)])


def kernel(user, item, U, I):
    user = user.astype(jnp.int32)
    item = item.astype(jnp.int32)
    Ut = U.T
    It = I.T
    mesh = plsc.VectorSubcoreMesh(core_axis_name="c", subcore_axis_name="s")
    k = functools.partial(
        pl.kernel,
        out_type=jax.ShapeDtypeStruct((BATCH,), jnp.float32),
        mesh=mesh,
        compiler_params=pltpu.CompilerParams(
            needs_layout_passes=False, use_tc_tiling_on_sc=True
        ),
        scratch_types=[
            pltpu.VMEM((BPW + L,), jnp.int32),
            pltpu.VMEM((BPW + L,), jnp.int32),
            pltpu.VMEM((EMBED, DEPTH * CW), jnp.float32),
            pltpu.VMEM((EMBED, DEPTH * CW), jnp.float32),
            pltpu.VMEM((BPW + L,), jnp.float32),
            pltpu.SemaphoreType.DMA((DEPTH,)),
        ],
    )(_mf_body)
    return k(user, item, Ut, It)
